# Initial kernel scaffold; baseline (speedup 1.0000x reference)
import functools

import jax
import jax.numpy as jnp
from jax import lax
from jax.experimental import pallas as pl
from jax.experimental.pallas import tpu as pltpu

N = 10000
E = 160000
F = 128
SSG = 16
NSUB = 10000
NSG = NSUB * SSG
ESG = 320000
E2 = 2 * ESG
HL = 128
DEPTH = SSG // 2

NB1 = 10
BN = N // NB1
NB2 = 125
BE = E // NB2
NB3 = 10
BG = NSUB // NB3


def _lrelu(x):
    return jnp.where(x >= 0, x, 0.01 * x)


def _dot(a, b):
    return jax.lax.dot_general(a, b, (((1,), (0,)), ((), ())),
                               preferred_element_type=jnp.float32)


def _k1_body(x_ref, s_ref, deg_ref,
             ws1, wn1, w0a1, w0b1, b01,
             ws2, wn2, w0a2, w0b2, b02,
             tsrc_ref, tdst_ref):
    x = x_ref[...]
    s = s_ref[...]
    degc = jnp.maximum(deg_ref[...], 1.0)
    for i, (ws, wn, w0a, w0b, b0) in enumerate(
            ((ws1, wn1, w0a1, w0b1, b01), (ws2, wn2, w0a2, w0b2, b02))):
        h = _lrelu(_dot(x, ws[...]) + _dot(s, wn[...]) / degc)
        tsrc_ref[:, i * HL:(i + 1) * HL] = _dot(h, w0a[...])
        tdst_ref[:, i * HL:(i + 1) * HL] = _dot(h, w0b[...]) + b0[...]


def _k1(x, S, deg, p1, p2):
    full = pl.BlockSpec((F, F), lambda i: (0, 0))
    vec = pl.BlockSpec((1, HL), lambda i: (0, 0))
    args = []
    specs = []
    for p in (p1, p2):
        args += [p['Ws'], p['Wn'], p['W0'][:F], p['W0'][F:2 * F],
                 p['b0'][None, :]]
        specs += [full, full, full, full, vec]
    return pl.pallas_call(
        _k1_body,
        grid=(NB1,),
        in_specs=[pl.BlockSpec((BN, F), lambda i: (i, 0)),
                  pl.BlockSpec((BN, F), lambda i: (i, 0)),
                  pl.BlockSpec((BN, 1), lambda i: (i, 0))] + specs,
        out_specs=[pl.BlockSpec((BN, 2 * HL), lambda i: (i, 0)),
                   pl.BlockSpec((BN, 2 * HL), lambda i: (i, 0))],
        out_shape=[jax.ShapeDtypeStruct((N, 2 * HL), jnp.float32),
                   jax.ShapeDtypeStruct((N, 2 * HL), jnp.float32)],
    )(x, S, deg, *args)


def _k2_body(g_ref, ang_ref, act_ref, gt_ref,
             wang1, wact1, w11, b11, wo1, bo1, wh1, bh1,
             wang2, wact2, w12, b12, wo2, bo2, wh2, bh2,
             e1_ref, e2_ref, loss_ref):
    pid = pl.program_id(0)

    @pl.when(pid == 0)
    def _():
        loss_ref[...] = jnp.zeros_like(loss_ref)

    ang = ang_ref[0, :][:, None]
    act = act_ref[0, :][:, None]
    gt = gt_ref[0, :][:, None]
    nets = ((wang1, wact1, w11, b11, wo1, bo1, wh1, bh1, e1_ref, 0),
            (wang2, wact2, w12, b12, wo2, bo2, wh2, bh2, e2_ref, 1))
    for (wang, wact, w1, b1, wo, bo, wh, bh, e_ref, i) in nets:
        z = (g_ref[:, i * HL:(i + 1) * HL]
             + ang * wang[...] + act * wact[...])
        y = _lrelu(_dot(_lrelu(z), w1[...]) + b1[...])
        e = _dot(y, wo[...]) + bo[...]
        e_ref[...] = e
        t = _dot(e, wh[...]) + bh[...]
        p = jax.nn.sigmoid(t)
        loss_ref[0, i] += jnp.sum((p - gt) ** 2)


def _k2(gsum, angles, actions, gt, p1, p2):
    full = pl.BlockSpec((F, F), lambda i: (0, 0))
    vec = pl.BlockSpec((1, HL), lambda i: (0, 0))
    col = pl.BlockSpec((F, 1), lambda i: (0, 0))
    one = pl.BlockSpec((1, 1), lambda i: (0, 0))
    args = []
    specs = []
    for p in (p1, p2):
        args += [p['W0'][2 * F][None, :], p['W0'][2 * F + 1][None, :],
                 p['W1'], p['b1'][None, :], p['Wout'], p['bout'][None, :],
                 p['Whead'], p['bhead'][None, :]]
        specs += [vec, vec, full, vec, full, vec, col, one]
    row = pl.BlockSpec((1, BE), lambda i: (i, 0))
    return pl.pallas_call(
        _k2_body,
        grid=(NB2,),
        in_specs=[pl.BlockSpec((BE, 2 * HL), lambda i: (i, 0)),
                  row, row, row] + specs,
        out_specs=[pl.BlockSpec((BE, HL), lambda i: (i, 0)),
                   pl.BlockSpec((BE, HL), lambda i: (i, 0)),
                   pl.BlockSpec((1, 2), lambda i: (0, 0))],
        out_shape=[jax.ShapeDtypeStruct((E, HL), jnp.float32),
                   jax.ShapeDtypeStruct((E, HL), jnp.float32),
                   jax.ShapeDtypeStruct((1, 2), jnp.float32)],
    )(gsum, angles.reshape(NB2, BE), actions.reshape(NB2, BE),
      gt.reshape(NB2, BE), *args)


def _k3_body(x1_ref, x2_ref,
             w01, b01, w11, b11, w21, b21,
             w02, b02, w12, b12, w22, b22,
             q1_ref, q2_ref):
    nets = ((x1_ref, w01, b01, w11, b11, w21, b21, q1_ref),
            (x2_ref, w02, b02, w12, b12, w22, b22, q2_ref))
    for (x_ref, w0, b0, w1, b1, w2, b2, q_ref) in nets:
        m = jnp.mean(x_ref[...], axis=1)
        h = _lrelu(_dot(m, w0[...]) + b0[...])
        h = _lrelu(_dot(h, w1[...]) + b1[...])
        q_ref[...] = _dot(h, w2[...]) + b2[...]


def _k3(X1, X2, v1, v2):
    full = pl.BlockSpec((F, HL), lambda i: (0, 0))
    vec = pl.BlockSpec((1, HL), lambda i: (0, 0))
    col = pl.BlockSpec((HL, 1), lambda i: (0, 0))
    one = pl.BlockSpec((1, 1), lambda i: (0, 0))
    args = []
    specs = []
    for v in (v1, v2):
        args += [v['W0'], v['b0'][None, :], v['W1'], v['b1'][None, :],
                 v['W2'], v['b2'][None, :]]
        specs += [full, vec, full, vec, col, one]
    x_spec = pl.BlockSpec((BG, SSG, F), lambda i: (i, 0, 0))
    return pl.pallas_call(
        _k3_body,
        grid=(NB3,),
        in_specs=[x_spec, x_spec] + specs,
        out_specs=[pl.BlockSpec((BG, 1), lambda i: (i, 0)),
                   pl.BlockSpec((BG, 1), lambda i: (i, 0))],
        out_shape=[jax.ShapeDtypeStruct((NSUB, 1), jnp.float32),
                   jax.ShapeDtypeStruct((NSUB, 1), jnp.float32)],
    )(X1.reshape(NSUB, SSG, F), X2.reshape(NSUB, SSG, F), *args)


def kernel(node_features, actions, edge_index, angles, sub_graphs,
           sep_subgraphs, gt_edges, post_input, p_gcn1_1, p_gcn2_1,
           p_gcn1_2_0, p_gcn2_2_0, p_value1, p_value2):
    x = node_features
    src, dst = edge_index[0], edge_index[1]

    S = jax.ops.segment_sum(jnp.take(x, src, axis=0), dst, num_segments=N)
    deg = jax.ops.segment_sum(jnp.ones((E,), jnp.float32), dst,
                              num_segments=N)[:, None]

    tsrc, tdst = _k1(x, S, deg, p_gcn1_1, p_gcn2_1)

    gsum = jnp.take(tsrc, src, axis=0) + jnp.take(tdst, dst, axis=0)

    e1, e2, loss = _k2(gsum, angles, actions, gt_edges, p_gcn1_1, p_gcn2_1)
    side = (loss[0, 0] + loss[0, 1]) / jnp.float32(E)

    X1 = jnp.take(e1, sub_graphs, axis=0)
    X2 = jnp.take(e2, sub_graphs, axis=0)

    src2 = jnp.concatenate([sep_subgraphs[0], sep_subgraphs[1]])
    dst2 = jnp.concatenate([sep_subgraphs[1], sep_subgraphs[0]])
    deg2raw = jax.ops.segment_sum(jnp.ones((E2,), jnp.float32), dst2,
                                  num_segments=NSG)
    deg2 = jnp.maximum(deg2raw, 1.0)[:, None]
    hb = (deg2raw > 0).astype(jnp.float32)[:, None]
    X = jnp.concatenate([X1, X2], axis=1)
    wcat = jnp.concatenate([p_gcn1_2_0['w'], p_gcn2_2_0['w']])
    bcat = jnp.concatenate([p_gcn1_2_0['b'], p_gcn2_2_0['b']])
    for _ in range(DEPTH):
        Ssub = jax.ops.segment_sum(jnp.take(X, src2, axis=0), dst2,
                                   num_segments=NSG)
        X = _lrelu(X + (Ssub * wcat) / deg2 + hb * bcat)

    q1, q2 = _k3(X[:, :F], X[:, F:], p_value1, p_value2)
    return (q1[:, 0], q2[:, 0], side / 4.0)


# speedup vs baseline: 1.3239x; 1.3239x over previous
import functools

import jax
import jax.numpy as jnp
from jax import lax
from jax.experimental import pallas as pl
from jax.experimental.pallas import tpu as pltpu

N = 10000
E = 160000
F = 128
SSG = 16
NSUB = 10000
NSG = NSUB * SSG
ESG = 320000
E2 = 2 * ESG
HL = 128
DEPTH = SSG // 2

NB1 = 10
BN = N // NB1
NB2 = 125
BE = E // NB2
NB3 = 10
BG = NSUB // NB3


def _lrelu(x):
    return jnp.where(x >= 0, x, 0.01 * x)


def _dot(a, b):
    return jax.lax.dot_general(a, b, (((1,), (0,)), ((), ())),
                               preferred_element_type=jnp.float32)


def _k1_body(x_ref, s_ref, deg_ref,
             ws1, wn1, w0a1, w0b1, b01,
             ws2, wn2, w0a2, w0b2, b02,
             tsrc_ref, tdst_ref):
    x = x_ref[...]
    s = s_ref[...]
    degc = jnp.maximum(deg_ref[...], 1.0)
    for i, (ws, wn, w0a, w0b, b0) in enumerate(
            ((ws1, wn1, w0a1, w0b1, b01), (ws2, wn2, w0a2, w0b2, b02))):
        h = _lrelu(_dot(x, ws[...]) + _dot(s, wn[...]) / degc)
        tsrc_ref[:, i * HL:(i + 1) * HL] = _dot(h, w0a[...])
        tdst_ref[:, i * HL:(i + 1) * HL] = _dot(h, w0b[...]) + b0[...]


def _k1(x, S, deg, p1, p2):
    full = pl.BlockSpec((F, F), lambda i: (0, 0))
    vec = pl.BlockSpec((1, HL), lambda i: (0, 0))
    args = []
    specs = []
    for p in (p1, p2):
        args += [p['Ws'], p['Wn'], p['W0'][:F], p['W0'][F:2 * F],
                 p['b0'][None, :]]
        specs += [full, full, full, full, vec]
    return pl.pallas_call(
        _k1_body,
        grid=(NB1,),
        in_specs=[pl.BlockSpec((BN, F), lambda i: (i, 0)),
                  pl.BlockSpec((BN, F), lambda i: (i, 0)),
                  pl.BlockSpec((BN, 1), lambda i: (i, 0))] + specs,
        out_specs=[pl.BlockSpec((BN, 2 * HL), lambda i: (i, 0)),
                   pl.BlockSpec((BN, 2 * HL), lambda i: (i, 0))],
        out_shape=[jax.ShapeDtypeStruct((N, 2 * HL), jnp.float32),
                   jax.ShapeDtypeStruct((N, 2 * HL), jnp.float32)],
    )(x, S, deg, *args)


def _k2_body(g_ref, ang_ref, act_ref, gt_ref,
             wang1, wact1, w11, b11, wo1, bo1, wh1, bh1,
             wang2, wact2, w12, b12, wo2, bo2, wh2, bh2,
             e1_ref, e2_ref, loss_ref):
    pid = pl.program_id(0)

    @pl.when(pid == 0)
    def _():
        loss_ref[...] = jnp.zeros_like(loss_ref)

    ang = ang_ref[0, 0, :][:, None]
    act = act_ref[0, 0, :][:, None]
    gt = gt_ref[0, 0, :][:, None]
    nets = ((wang1, wact1, w11, b11, wo1, bo1, wh1, bh1, e1_ref, 0),
            (wang2, wact2, w12, b12, wo2, bo2, wh2, bh2, e2_ref, 1))
    losses = []
    for (wang, wact, w1, b1, wo, bo, wh, bh, e_ref, i) in nets:
        z = (g_ref[:, i * HL:(i + 1) * HL]
             + ang * wang[...] + act * wact[...])
        y = _lrelu(_dot(_lrelu(z), w1[...]) + b1[...])
        e = _dot(y, wo[...]) + bo[...]
        e_ref[...] = e
        t = _dot(e, wh[...]) + bh[...]
        p = jax.nn.sigmoid(t)
        losses.append(jnp.sum((p - gt) ** 2))
    loss_ref[...] += jnp.stack(losses)[None, :]


def _k2(gsum, angles, actions, gt, p1, p2):
    full = pl.BlockSpec((F, F), lambda i: (0, 0))
    vec = pl.BlockSpec((1, HL), lambda i: (0, 0))
    col = pl.BlockSpec((F, 1), lambda i: (0, 0))
    one = pl.BlockSpec((1, 1), lambda i: (0, 0))
    args = []
    specs = []
    for p in (p1, p2):
        args += [p['W0'][2 * F][None, :], p['W0'][2 * F + 1][None, :],
                 p['W1'], p['b1'][None, :], p['Wout'], p['bout'][None, :],
                 p['Whead'], p['bhead'][None, :]]
        specs += [vec, vec, full, vec, full, vec, col, one]
    row = pl.BlockSpec((1, 1, BE), lambda i: (i, 0, 0))
    return pl.pallas_call(
        _k2_body,
        grid=(NB2,),
        in_specs=[pl.BlockSpec((BE, 2 * HL), lambda i: (i, 0)),
                  row, row, row] + specs,
        out_specs=[pl.BlockSpec((BE, HL), lambda i: (i, 0)),
                   pl.BlockSpec((BE, HL), lambda i: (i, 0)),
                   pl.BlockSpec((1, 2), lambda i: (0, 0))],
        out_shape=[jax.ShapeDtypeStruct((E, HL), jnp.float32),
                   jax.ShapeDtypeStruct((E, HL), jnp.float32),
                   jax.ShapeDtypeStruct((1, 2), jnp.float32)],
    )(gsum, angles.reshape(NB2, 1, BE), actions.reshape(NB2, 1, BE),
      gt.reshape(NB2, 1, BE), *args)


def _k3_body(x1_ref, x2_ref,
             w01, b01, w11, b11, w21, b21,
             w02, b02, w12, b12, w22, b22,
             q1_ref, q2_ref):
    nets = ((x1_ref, w01, b01, w11, b11, w21, b21, q1_ref),
            (x2_ref, w02, b02, w12, b12, w22, b22, q2_ref))
    for (x_ref, w0, b0, w1, b1, w2, b2, q_ref) in nets:
        m = jnp.mean(x_ref[...], axis=1)
        h = _lrelu(_dot(m, w0[...]) + b0[...])
        h = _lrelu(_dot(h, w1[...]) + b1[...])
        q_ref[...] = _dot(h, w2[...]) + b2[...]


def _k3(X1, X2, v1, v2):
    full = pl.BlockSpec((F, HL), lambda i: (0, 0))
    vec = pl.BlockSpec((1, HL), lambda i: (0, 0))
    col = pl.BlockSpec((HL, 1), lambda i: (0, 0))
    one = pl.BlockSpec((1, 1), lambda i: (0, 0))
    args = []
    specs = []
    for v in (v1, v2):
        args += [v['W0'], v['b0'][None, :], v['W1'], v['b1'][None, :],
                 v['W2'], v['b2'][None, :]]
        specs += [full, vec, full, vec, col, one]
    x_spec = pl.BlockSpec((BG, SSG, F), lambda i: (i, 0, 0))
    return pl.pallas_call(
        _k3_body,
        grid=(NB3,),
        in_specs=[x_spec, x_spec] + specs,
        out_specs=[pl.BlockSpec((BG, 1), lambda i: (i, 0)),
                   pl.BlockSpec((BG, 1), lambda i: (i, 0))],
        out_shape=[jax.ShapeDtypeStruct((NSUB, 1), jnp.float32),
                   jax.ShapeDtypeStruct((NSUB, 1), jnp.float32)],
    )(X1.reshape(NSUB, SSG, F), X2.reshape(NSUB, SSG, F), *args)


def kernel(node_features, actions, edge_index, angles, sub_graphs,
           sep_subgraphs, gt_edges, post_input, p_gcn1_1, p_gcn2_1,
           p_gcn1_2_0, p_gcn2_2_0, p_value1, p_value2):
    x = node_features
    src, dst = edge_index[0], edge_index[1]

    S = jax.ops.segment_sum(jnp.take(x, src, axis=0), dst, num_segments=N)
    deg = jax.ops.segment_sum(jnp.ones((E,), jnp.float32), dst,
                              num_segments=N)[:, None]

    tsrc, tdst = _k1(x, S, deg, p_gcn1_1, p_gcn2_1)

    gsum = jnp.take(tsrc, src, axis=0) + jnp.take(tdst, dst, axis=0)

    e1, e2, loss = _k2(gsum, angles, actions, gt_edges, p_gcn1_1, p_gcn2_1)
    side = (loss[0, 0] + loss[0, 1]) / jnp.float32(E)

    X1 = jnp.take(e1, sub_graphs, axis=0)
    X2 = jnp.take(e2, sub_graphs, axis=0)

    src2 = jnp.concatenate([sep_subgraphs[0], sep_subgraphs[1]])
    dst2 = jnp.concatenate([sep_subgraphs[1], sep_subgraphs[0]])
    deg2raw = jax.ops.segment_sum(jnp.ones((E2,), jnp.float32), dst2,
                                  num_segments=NSG)
    deg2 = jnp.maximum(deg2raw, 1.0)[:, None]
    hb = (deg2raw > 0).astype(jnp.float32)[:, None]
    X = jnp.concatenate([X1, X2], axis=1)
    wcat = jnp.concatenate([p_gcn1_2_0['w'], p_gcn2_2_0['w']])
    bcat = jnp.concatenate([p_gcn1_2_0['b'], p_gcn2_2_0['b']])
    for _ in range(DEPTH):
        Ssub = jax.ops.segment_sum(jnp.take(X, src2, axis=0), dst2,
                                   num_segments=NSG)
        X = _lrelu(X + (Ssub * wcat) / deg2 + hb * bcat)

    q1, q2 = _k3(X[:, :F], X[:, F:], p_value1, p_value2)
    return (q1[:, 0], q2[:, 0], side / 4.0)


# speedup vs baseline: 1.3620x; 1.0287x over previous
import functools

import jax
import jax.numpy as jnp
from jax import lax
from jax.experimental import pallas as pl
from jax.experimental.pallas import tpu as pltpu
from jax.experimental.pallas import tpu_sc as plsc

N = 10000
E = 160000
F = 128
SSG = 16
NSUB = 10000
NSG = NSUB * SSG
ESG = 320000
E2 = 2 * ESG
HL = 128
DEPTH = SSG // 2

NB1 = 10
BN = N // NB1
NB2 = 125
BE = E // NB2
NB3 = 10
BG = NSUB // NB3


def _lrelu(x):
    return jnp.where(x >= 0, x, 0.01 * x)


def _dot(a, b):
    return jax.lax.dot_general(a, b, (((1,), (0,)), ((), ())),
                               preferred_element_type=jnp.float32)


NP = 10240
CP = 128
EB = 512
EBLK = 10
EP = 32 * EBLK * EB


CH1 = NP // 2
AR1 = CH1 + 256


def _sc1_body(xpad_hbm, src_hbm, dst_hbm, zeros_hbm,
              out_hbm, dout_hbm,
              idx_s, idx_d, rows_v, ones_v, dzero, dbuf, sem, acc, dacc):
    c = lax.axis_index("c")
    s = lax.axis_index("s")
    w = c * 16 + s
    for i in range(EB // 16):
        ones_v[pl.ds(i * 16, 16)] = jnp.ones((16,), jnp.float32)
    zlen = AR1 // 16
    for i in range(zlen // 16):
        dzero[pl.ds(i * 16, 16)] = jnp.zeros((16,), jnp.float32)
    for ch in range(2):
        pltpu.sync_copy(zeros_hbm.at[pl.ds(s * (AR1 // 16), AR1 // 16)],
                        acc.at[pl.ds(s * (AR1 // 16), AR1 // 16)])
        pltpu.sync_copy(dzero.at[pl.ds(0, zlen)],
                        dacc.at[pl.ds(s * zlen, zlen)])
        plsc.subcore_barrier()
        for j in range(EBLK):
            pltpu.sync_copy(src_hbm.at[w, pl.ds(j * EB, EB)], idx_s)
            pltpu.sync_copy(dst_hbm.at[ch, w, pl.ds(j * EB, EB)], idx_d)
            pltpu.async_copy(xpad_hbm.at[idx_s], rows_v, sem).wait()
            pltpu.sync_copy(rows_v, acc.at[idx_d], add=True)
            pltpu.sync_copy(ones_v, dacc.at[idx_d], add=True)
        plsc.subcore_barrier()
        pltpu.sync_copy(acc.at[pl.ds(s * (CH1 // 16), CH1 // 16)],
                        out_hbm.at[c, pl.ds(ch * CH1 + s * (CH1 // 16),
                                            CH1 // 16)])

        dseg = CH1 // 16
        pltpu.sync_copy(dacc.at[pl.ds(s * dseg, dseg)],
                        dbuf.at[pl.ds(0, dseg)])
        pltpu.sync_copy(dbuf.at[pl.ds(0, dseg)],
                        dout_hbm.at[pl.ds(c * NP + ch * CH1 + s * dseg, dseg)])
        plsc.subcore_barrier()


@jax.jit
def _sc1(xpad, srcb, dstb, zeros):
    mesh = plsc.VectorSubcoreMesh(core_axis_name="c", subcore_axis_name="s")
    return pl.kernel(
        _sc1_body,
        mesh=mesh,
        out_type=[jax.ShapeDtypeStruct((2, NP, CP), jnp.float32),
                  jax.ShapeDtypeStruct((2 * NP,), jnp.float32)],
        scratch_types=[
            pltpu.VMEM((EB,), jnp.int32),
            pltpu.VMEM((EB,), jnp.int32),
            pltpu.VMEM((EB, CP), jnp.float32),
            pltpu.VMEM((EB,), jnp.float32),
            pltpu.VMEM((AR1 // 16,), jnp.float32),
            pltpu.VMEM((CH1 // 16,), jnp.float32),
            pltpu.SemaphoreType.DMA,
            pltpu.VMEM_SHARED((AR1, CP), jnp.float32),
            pltpu.VMEM_SHARED((AR1,), jnp.float32),
        ],
    )(xpad, srcb, dstb, zeros)


def _k1_body(x_ref, s0_ref, s1_ref, d0_ref, d1_ref,
             ws1, wn1, w0a1, w0b1, b01,
             ws2, wn2, w0a2, w0b2, b02,
             tsrc_ref, tdst_ref):
    x = x_ref[...]
    s = s0_ref[...] + s1_ref[...]
    degc = jnp.maximum(d0_ref[...] + d1_ref[...], 1.0)
    for i, (ws, wn, w0a, w0b, b0) in enumerate(
            ((ws1, wn1, w0a1, w0b1, b01), (ws2, wn2, w0a2, w0b2, b02))):
        h = _lrelu(_dot(x, ws[...]) + _dot(s, wn[...]) / degc)
        tsrc_ref[:, i * HL:(i + 1) * HL] = _dot(h, w0a[...])
        tdst_ref[:, i * HL:(i + 1) * HL] = _dot(h, w0b[...]) + b0[...]


def _k1(x, S0, S1, d0, d1, p1, p2):
    full = pl.BlockSpec((F, F), lambda i: (0, 0))
    vec = pl.BlockSpec((1, HL), lambda i: (0, 0))
    args = []
    specs = []
    for p in (p1, p2):
        args += [p['Ws'], p['Wn'], p['W0'][:F], p['W0'][F:2 * F],
                 p['b0'][None, :]]
        specs += [full, full, full, full, vec]
    return pl.pallas_call(
        _k1_body,
        grid=(NB1,),
        in_specs=[pl.BlockSpec((BN, F), lambda i: (i, 0)),
                  pl.BlockSpec((BN, CP), lambda i: (i, 0)),
                  pl.BlockSpec((BN, CP), lambda i: (i, 0)),
                  pl.BlockSpec((BN, 1), lambda i: (i, 0)),
                  pl.BlockSpec((BN, 1), lambda i: (i, 0))] + specs,
        out_specs=[pl.BlockSpec((BN, 2 * HL), lambda i: (i, 0)),
                   pl.BlockSpec((BN, 2 * HL), lambda i: (i, 0))],
        out_shape=[jax.ShapeDtypeStruct((N, 2 * HL), jnp.float32),
                   jax.ShapeDtypeStruct((N, 2 * HL), jnp.float32)],
    )(x, S0, S1, d0, d1, *args)


def _k2_body(g_ref, ang_ref, act_ref, gt_ref,
             wang1, wact1, w11, b11, wo1, bo1, wh1, bh1,
             wang2, wact2, w12, b12, wo2, bo2, wh2, bh2,
             e1_ref, e2_ref, loss_ref):
    pid = pl.program_id(0)

    @pl.when(pid == 0)
    def _():
        loss_ref[...] = jnp.zeros_like(loss_ref)

    ang = ang_ref[0, 0, :][:, None]
    act = act_ref[0, 0, :][:, None]
    gt = gt_ref[0, 0, :][:, None]
    nets = ((wang1, wact1, w11, b11, wo1, bo1, wh1, bh1, e1_ref, 0),
            (wang2, wact2, w12, b12, wo2, bo2, wh2, bh2, e2_ref, 1))
    losses = []
    for (wang, wact, w1, b1, wo, bo, wh, bh, e_ref, i) in nets:
        z = (g_ref[:, i * HL:(i + 1) * HL]
             + ang * wang[...] + act * wact[...])
        y = _lrelu(_dot(_lrelu(z), w1[...]) + b1[...])
        e = _dot(y, wo[...]) + bo[...]
        e_ref[...] = e
        t = _dot(e, wh[...]) + bh[...]
        p = jax.nn.sigmoid(t)
        losses.append(jnp.sum((p - gt) ** 2))
    loss_ref[...] += jnp.stack(losses)[None, :]


def _k2(gsum, angles, actions, gt, p1, p2):
    full = pl.BlockSpec((F, F), lambda i: (0, 0))
    vec = pl.BlockSpec((1, HL), lambda i: (0, 0))
    col = pl.BlockSpec((F, 1), lambda i: (0, 0))
    one = pl.BlockSpec((1, 1), lambda i: (0, 0))
    args = []
    specs = []
    for p in (p1, p2):
        args += [p['W0'][2 * F][None, :], p['W0'][2 * F + 1][None, :],
                 p['W1'], p['b1'][None, :], p['Wout'], p['bout'][None, :],
                 p['Whead'], p['bhead'][None, :]]
        specs += [vec, vec, full, vec, full, vec, col, one]
    row = pl.BlockSpec((1, 1, BE), lambda i: (i, 0, 0))
    return pl.pallas_call(
        _k2_body,
        grid=(NB2,),
        in_specs=[pl.BlockSpec((BE, 2 * HL), lambda i: (i, 0)),
                  row, row, row] + specs,
        out_specs=[pl.BlockSpec((BE, HL), lambda i: (i, 0)),
                   pl.BlockSpec((BE, HL), lambda i: (i, 0)),
                   pl.BlockSpec((1, 2), lambda i: (0, 0))],
        out_shape=[jax.ShapeDtypeStruct((E, HL), jnp.float32),
                   jax.ShapeDtypeStruct((E, HL), jnp.float32),
                   jax.ShapeDtypeStruct((1, 2), jnp.float32)],
    )(gsum, angles.reshape(NB2, 1, BE), actions.reshape(NB2, 1, BE),
      gt.reshape(NB2, 1, BE), *args)


def _k3_body(x1_ref, x2_ref,
             w01, b01, w11, b11, w21, b21,
             w02, b02, w12, b12, w22, b22,
             q1_ref, q2_ref):
    nets = ((x1_ref, w01, b01, w11, b11, w21, b21, q1_ref),
            (x2_ref, w02, b02, w12, b12, w22, b22, q2_ref))
    for (x_ref, w0, b0, w1, b1, w2, b2, q_ref) in nets:
        m = jnp.mean(x_ref[...], axis=1)
        h = _lrelu(_dot(m, w0[...]) + b0[...])
        h = _lrelu(_dot(h, w1[...]) + b1[...])
        q_ref[...] = _dot(h, w2[...]) + b2[...]


def _k3(X1, X2, v1, v2):
    full = pl.BlockSpec((F, HL), lambda i: (0, 0))
    vec = pl.BlockSpec((1, HL), lambda i: (0, 0))
    col = pl.BlockSpec((HL, 1), lambda i: (0, 0))
    one = pl.BlockSpec((1, 1), lambda i: (0, 0))
    args = []
    specs = []
    for v in (v1, v2):
        args += [v['W0'], v['b0'][None, :], v['W1'], v['b1'][None, :],
                 v['W2'], v['b2'][None, :]]
        specs += [full, vec, full, vec, col, one]
    x_spec = pl.BlockSpec((BG, SSG, F), lambda i: (i, 0, 0))
    return pl.pallas_call(
        _k3_body,
        grid=(NB3,),
        in_specs=[x_spec, x_spec] + specs,
        out_specs=[pl.BlockSpec((BG, 1), lambda i: (i, 0)),
                   pl.BlockSpec((BG, 1), lambda i: (i, 0))],
        out_shape=[jax.ShapeDtypeStruct((NSUB, 1), jnp.float32),
                   jax.ShapeDtypeStruct((NSUB, 1), jnp.float32)],
    )(X1.reshape(NSUB, SSG, F), X2.reshape(NSUB, SSG, F), *args)


def kernel(node_features, actions, edge_index, angles, sub_graphs,
           sep_subgraphs, gt_edges, post_input, p_gcn1_1, p_gcn2_1,
           p_gcn1_2_0, p_gcn2_2_0, p_value1, p_value2):
    x = node_features
    src, dst = edge_index[0], edge_index[1]

    xpad = jnp.zeros((NP, CP), jnp.float32).at[:N].set(x)
    pad_idx = N + (jnp.arange(EP - E, dtype=jnp.int32) % 8)
    srcb = jnp.concatenate([src, pad_idx]).reshape(32, EBLK * EB)
    dst_all = jnp.concatenate([dst, pad_idx])
    dumm = CH1 + (jnp.arange(EP, dtype=jnp.int32) % 8)
    dstb = jnp.stack([
        jnp.where((dst_all >= ch * CH1) & (dst_all < (ch + 1) * CH1),
                  dst_all - ch * CH1, dumm).reshape(32, EBLK * EB)
        for ch in range(2)])
    Sp, Dpf = _sc1(xpad, srcb, dstb, jnp.zeros((AR1, CP), jnp.float32))
    Dp = Dpf.reshape(2, NP)

    tsrc, tdst = _k1(x, Sp[0], Sp[1], Dp[0][:, None], Dp[1][:, None],
                     p_gcn1_1, p_gcn2_1)

    gsum = jnp.take(tsrc, src, axis=0) + jnp.take(tdst, dst, axis=0)

    e1, e2, loss = _k2(gsum, angles, actions, gt_edges, p_gcn1_1, p_gcn2_1)
    side = (loss[0, 0] + loss[0, 1]) / jnp.float32(E)

    X1 = jnp.take(e1, sub_graphs, axis=0)
    X2 = jnp.take(e2, sub_graphs, axis=0)

    src2 = jnp.concatenate([sep_subgraphs[0], sep_subgraphs[1]])
    dst2 = jnp.concatenate([sep_subgraphs[1], sep_subgraphs[0]])
    deg2raw = jax.ops.segment_sum(jnp.ones((E2,), jnp.float32), dst2,
                                  num_segments=NSG)
    deg2 = jnp.maximum(deg2raw, 1.0)[:, None]
    hb = (deg2raw > 0).astype(jnp.float32)[:, None]
    X = jnp.concatenate([X1, X2], axis=1)
    wcat = jnp.concatenate([p_gcn1_2_0['w'], p_gcn2_2_0['w']])
    bcat = jnp.concatenate([p_gcn1_2_0['b'], p_gcn2_2_0['b']])
    for _ in range(DEPTH):
        Ssub = jax.ops.segment_sum(jnp.take(X, src2, axis=0), dst2,
                                   num_segments=NSG)
        X = _lrelu(X + (Ssub * wcat) / deg2 + hb * bcat)

    q1, q2 = _k3(X[:, :F], X[:, F:], p_value1, p_value2)
    return (q1[:, 0], q2[:, 0], side / 4.0)
